# Initial kernel scaffold; baseline (speedup 1.0000x reference)
import jax
import jax.numpy as jnp
from jax.experimental import pallas as pl


def kernel(xyz1, xyz2, feat1, feat2, W1, b1, g1, be1, W2, b2, g2, be2, W3, b3, g3, be3):
    raise NotImplementedError("write your pallas kernel here")


# speedup vs baseline: 12.6221x; 12.6221x over previous
import functools

import jax
import jax.numpy as jnp
from jax import lax
from jax.experimental import pallas as pl
from jax.experimental.pallas import tpu as pltpu
from jax.experimental.pallas import tpu_sc as plsc

B, N, S, C = 4, 1024, 64, 64
EPS = 1e-3
CNT = float(B * N * S)
NQ = B * N
SP = S // 2
QB = 64
HI0 = 0x7F800000


def _k0_body(x1_ref, x2_ref, f1_ref, f2_ref, w1a_ref, w1b_ref, w1c_ref, b1_ref,
             gl_ref, gr_ref, h_ref):
    x1 = x1_ref[0]
    x2 = x2_ref[0]
    f1 = f1_ref[0]
    f2 = f2_ref[0]
    dn = (((0,), (0,)), ((), ()))
    g = (lax.dot_general(f2, w1b_ref[...], dn, preferred_element_type=jnp.float32)
         + lax.dot_general(x2, w1a_ref[...], dn, preferred_element_type=jnp.float32))
    h = (lax.dot_general(f1, w1c_ref[...], dn, preferred_element_type=jnp.float32)
         - lax.dot_general(x1, w1a_ref[...], dn, preferred_element_type=jnp.float32)
         + b1_ref[...])
    z = jnp.zeros((N, 64), jnp.float32)
    gl_ref[0] = jnp.concatenate([g, z], axis=1)
    gr_ref[0] = jnp.concatenate([z, g], axis=1)
    h_ref[0] = jnp.concatenate([h, h], axis=1)


def _cumsum0(x):
    xf = x.astype(jnp.float32)
    tri = (lax.broadcasted_iota(jnp.int32, (128, 128), 1)
           <= lax.broadcasted_iota(jnp.int32, (128, 128), 0)).astype(jnp.float32)
    parts = []
    off = jnp.zeros((1, 128), jnp.float32)
    for k in range(8):
        blk = lax.slice(xf, (k * 128, 0), (k * 128 + 128, 128))
        loc = lax.dot_general(tri, blk, (((1,), (0,)), ((), ())),
                              preferred_element_type=jnp.float32)
        parts.append(loc + off)
        off = off + lax.slice(loc, (127, 0), (128, 128))
    return jnp.concatenate(parts, axis=0).astype(jnp.int32)


def _ka_body(d_ref, r_ref):
    di = lax.bitcast_convert_type(d_ref[0], jnp.int32)

    def body(_, carry):
        lo, hi = carry
        mid = lo + ((hi - lo) >> 1)
        cnt = jnp.sum((di <= mid).astype(jnp.int32), axis=0, keepdims=True)
        ok = cnt >= S
        return jnp.where(ok, lo, mid), jnp.where(ok, mid, hi)

    lo0 = jnp.full((1, 128), -1, jnp.int32)
    hi0 = jnp.full((1, 128), HI0, jnp.int32)
    _, t = lax.fori_loop(0, 31, body, (lo0, hi0))

    lt = (di < t).astype(jnp.int32)
    c_lt = jnp.sum(lt, axis=0, keepdims=True)
    r_need = S - c_lt
    eq = (di == t).astype(jnp.int32)
    cs_eq = _cumsum0(eq)
    sel = lt + eq * ((cs_eq - eq) < r_need).astype(jnp.int32)
    r_ref[0] = _cumsum0(sel)


def _kc1_body(y_ref, h_ref, st_ref):
    i = pl.program_id(0)
    y = y_ref[...] + h_ref[...][:, None, :]
    s0 = jnp.sum(y, axis=(0, 1), keepdims=True)[0]
    s1 = jnp.sum(y * y, axis=(0, 1), keepdims=True)[0]
    st = jnp.concatenate([s0, s1], axis=0)

    @pl.when(i == 0)
    def _():
        st_ref[...] = st

    @pl.when(i > 0)
    def _():
        st_ref[...] += st


def _fold_stats(st):
    sf = st[:, :64] + st[:, 64:]
    sf = jnp.concatenate([sf, sf], axis=1)
    mean = sf[0:1] / CNT
    var = sf[1:2] / CNT - mean * mean
    inv = lax.rsqrt(var + EPS)
    return mean, inv


def _kc2_body(y_ref, h_ref, st_ref, p_ref, w_ref, b_ref, o_ref, st2_ref):
    i = pl.program_id(0)
    mean, inv = _fold_stats(st_ref[...])
    y = y_ref[...] + h_ref[...][:, None, :]
    x = jnp.reshape(y, (QB * SP, 128))
    x = jnp.maximum((x - mean) * inv * p_ref[0:1] + p_ref[1:2], 0.0)
    y2 = lax.dot_general(x, w_ref[...], (((1,), (0,)), ((), ())),
                         preferred_element_type=jnp.float32) + b_ref[...]
    o_ref[...] = jnp.reshape(y2, (QB, SP, 128))
    s0 = jnp.sum(y2, axis=0, keepdims=True)
    s1 = jnp.sum(y2 * y2, axis=0, keepdims=True)
    st = jnp.concatenate([s0, s1], axis=0)

    @pl.when(i == 0)
    def _():
        st2_ref[...] = st

    @pl.when(i > 0)
    def _():
        st2_ref[...] += st


def _kc3_body(y2_ref, st_ref, p_ref, w_ref, b_ref, mx_ref, mn_ref, st3_ref):
    i = pl.program_id(0)
    mean, inv = _fold_stats(st_ref[...])
    x = jnp.reshape(y2_ref[...], (QB * SP, 128))
    x = jnp.maximum((x - mean) * inv * p_ref[0:1] + p_ref[1:2], 0.0)
    y3 = lax.dot_general(x, w_ref[...], (((1,), (0,)), ((), ())),
                         preferred_element_type=jnp.float32) + b_ref[...]
    s0 = jnp.sum(y3, axis=0, keepdims=True)
    s1 = jnp.sum(y3 * y3, axis=0, keepdims=True)
    st = jnp.concatenate([s0, s1], axis=0)
    y3r = jnp.reshape(y3, (QB, SP, 128))
    mx = jnp.max(y3r, axis=1)
    mn = jnp.min(y3r, axis=1)
    mx_ref[...] = jnp.maximum(mx[:, :64], mx[:, 64:])
    mn_ref[...] = jnp.minimum(mn[:, :64], mn[:, 64:])

    @pl.when(i == 0)
    def _():
        st3_ref[...] = st

    @pl.when(i > 0)
    def _():
        st3_ref[...] += st


def _kc4_body(mx_ref, mn_ref, st_ref, g_ref, be_ref, o_ref):
    st = st_ref[...]
    sf = st[:, :64] + st[:, 64:]
    mean = sf[0:1] / CNT
    var = sf[1:2] / CNT - mean * mean
    inv = lax.rsqrt(var + EPS)
    g = g_ref[...]
    m = jnp.where(g > 0.0, mx_ref[...], mn_ref[...])
    o_ref[...] = jnp.maximum((m - mean) * inv * g + be_ref[...], 0.0)


def _tc_pipeline(xyz1p, xyz2p, feat1, feat2, w1a, w1b, w1c, b1r, dist):
    gl, gr, h = pl.pallas_call(
        _k0_body,
        grid=(B,),
        in_specs=[
            pl.BlockSpec((1, 8, N), lambda b: (b, 0, 0)),
            pl.BlockSpec((1, 8, N), lambda b: (b, 0, 0)),
            pl.BlockSpec((1, C, N), lambda b: (b, 0, 0)),
            pl.BlockSpec((1, C, N), lambda b: (b, 0, 0)),
            pl.BlockSpec((8, 64), lambda b: (0, 0)),
            pl.BlockSpec((C, 64), lambda b: (0, 0)),
            pl.BlockSpec((C, 64), lambda b: (0, 0)),
            pl.BlockSpec((1, 64), lambda b: (0, 0)),
        ],
        out_specs=[
            pl.BlockSpec((1, N, 128), lambda b: (b, 0, 0)),
            pl.BlockSpec((1, N, 128), lambda b: (b, 0, 0)),
            pl.BlockSpec((1, N, 128), lambda b: (b, 0, 0)),
        ],
        out_shape=[
            jax.ShapeDtypeStruct((B, N, 128), jnp.float32),
            jax.ShapeDtypeStruct((B, N, 128), jnp.float32),
            jax.ShapeDtypeStruct((B, N, 128), jnp.float32),
        ],
    )(xyz1p, xyz2p, feat1, feat2, w1a, w1b, w1c, b1r)

    r = pl.pallas_call(
        _ka_body,
        grid=(B, 8),
        in_specs=[
            pl.BlockSpec((1, N, 128), lambda b, q: (b, 0, q)),
        ],
        out_specs=pl.BlockSpec((1, N, 128), lambda b, q: (b, 0, q)),
        out_shape=jax.ShapeDtypeStruct((B, N, N), jnp.int32),
    )(dist)
    return gl, gr, h, r


def _tc_tail(y1g, hdup, p1, b2r, w2d, p2, b3r, w3d, g3r, be3r):
    grid = (NQ // QB,)
    st1 = pl.pallas_call(
        _kc1_body,
        grid=grid,
        in_specs=[
            pl.BlockSpec((QB, SP, 128), lambda i: (i, 0, 0)),
            pl.BlockSpec((QB, 128), lambda i: (i, 0)),
        ],
        out_specs=pl.BlockSpec((2, 128), lambda i: (0, 0)),
        out_shape=jax.ShapeDtypeStruct((2, 128), jnp.float32),
    )(y1g, hdup)

    y2, st2 = pl.pallas_call(
        _kc2_body,
        grid=grid,
        in_specs=[
            pl.BlockSpec((QB, SP, 128), lambda i: (i, 0, 0)),
            pl.BlockSpec((QB, 128), lambda i: (i, 0)),
            pl.BlockSpec((2, 128), lambda i: (0, 0)),
            pl.BlockSpec((2, 128), lambda i: (0, 0)),
            pl.BlockSpec((128, 128), lambda i: (0, 0)),
            pl.BlockSpec((1, 128), lambda i: (0, 0)),
        ],
        out_specs=[
            pl.BlockSpec((QB, SP, 128), lambda i: (i, 0, 0)),
            pl.BlockSpec((2, 128), lambda i: (0, 0)),
        ],
        out_shape=[
            jax.ShapeDtypeStruct((NQ, SP, 128), jnp.float32),
            jax.ShapeDtypeStruct((2, 128), jnp.float32),
        ],
    )(y1g, hdup, st1, p1, w2d, b2r)

    mx, mn, st3 = pl.pallas_call(
        _kc3_body,
        grid=grid,
        in_specs=[
            pl.BlockSpec((QB, SP, 128), lambda i: (i, 0, 0)),
            pl.BlockSpec((2, 128), lambda i: (0, 0)),
            pl.BlockSpec((2, 128), lambda i: (0, 0)),
            pl.BlockSpec((128, 128), lambda i: (0, 0)),
            pl.BlockSpec((1, 128), lambda i: (0, 0)),
        ],
        out_specs=[
            pl.BlockSpec((QB, 64), lambda i: (i, 0)),
            pl.BlockSpec((QB, 64), lambda i: (i, 0)),
            pl.BlockSpec((2, 128), lambda i: (0, 0)),
        ],
        out_shape=[
            jax.ShapeDtypeStruct((NQ, 64), jnp.float32),
            jax.ShapeDtypeStruct((NQ, 64), jnp.float32),
            jax.ShapeDtypeStruct((2, 128), jnp.float32),
        ],
    )(y2, st2, p2, w3d, b3r)

    out = pl.pallas_call(
        _kc4_body,
        grid=(1,),
        in_specs=[
            pl.BlockSpec((NQ, 64), lambda i: (0, 0)),
            pl.BlockSpec((NQ, 64), lambda i: (0, 0)),
            pl.BlockSpec((2, 128), lambda i: (0, 0)),
            pl.BlockSpec((1, 64), lambda i: (0, 0)),
            pl.BlockSpec((1, 64), lambda i: (0, 0)),
        ],
        out_specs=pl.BlockSpec((NQ, 64), lambda i: (0, 0)),
        out_shape=jax.ShapeDtypeStruct((NQ, 64), jnp.float32),
    )(mx, mn, st3, g3r, be3r)
    return out


def kernel(xyz1, xyz2, feat1, feat2, W1, b1, g1, be1, W2, b2, g2, be2, W3, b3, g3, be3):
    f32 = jnp.float32
    pad = jnp.zeros((B, 5, N), f32)
    xyz1p = jnp.concatenate([xyz1, pad], axis=1)
    xyz2p = jnp.concatenate([xyz2, pad], axis=1)
    w1a = jnp.concatenate([W1[:, :3], jnp.zeros((64, 5), f32)], axis=1).T
    w1b = W1[:, 3:67].T
    w1c = W1[:, 67:131].T
    b1r = b1[None, :]

    xx = jnp.sum(xyz2 ** 2, axis=1)[:, :, None]
    yy = jnp.sum(xyz1 ** 2, axis=1)[:, None, :]
    dist = xx + yy - 2.0 * jnp.einsum('bcn,bcm->bnm', xyz2, xyz1)
    dist = jnp.clip(jnp.nan_to_num(dist), 0.0, None)

    gl, gr, hdup, cs = _tc_pipeline(xyz1p, xyz2p, feat1, feat2, w1a, w1b, w1c,
                                    b1r, dist)

    cst = jnp.reshape(jnp.transpose(cs, (0, 2, 1)), (NQ, N))
    glf = jnp.reshape(gl, (NQ, 128))
    grf = jnp.reshape(gr, (NQ, 128))
    y1g = _sc_gather(cst, glf, grf)

    y1g = jnp.reshape(y1g, (NQ, SP, 128))
    hdupf = jnp.reshape(hdup, (NQ, 128))

    def dup(v):
        return jnp.concatenate([v, v])[None, :]

    p1 = jnp.concatenate([dup(g1), dup(be1)], axis=0)
    p2 = jnp.concatenate([dup(g2), dup(be2)], axis=0)
    w2t = W2.T
    w3t = W3.T
    zz = jnp.zeros((64, 64), f32)
    w2d = jnp.block([[w2t, zz], [zz, w2t]])
    w3d = jnp.block([[w3t, zz], [zz, w3t]])
    b2r = dup(b2)
    b3r = dup(b3)

    out = _tc_tail(y1g, hdupf, p1, b2r, w2d, p2, b3r, w3d, g3[None, :], be3[None, :])
    return jnp.transpose(jnp.reshape(out, (B, N, 64)), (0, 2, 1))


def _sc_body(cs_hbm, gl_hbm, gr_hbm, out_hbm, rstage, idxe, idxo, rows, sem):
    wid = lax.axis_index("s") * 2 + lax.axis_index("c")
    qbase = wid * 128

    def sixteen(sb, c0):
        base_q = qbase + sb * 16
        jb = ((base_q >> 10) << 10)
        pltpu.sync_copy(cs_hbm.at[pl.ds(base_q, 16)], rstage)

        def perq(qi, c1):
            qf = jnp.full((16,), qi, jnp.int32)

            def slotv(sv, c2):
                target = lax.iota(jnp.int32, 16) + (sv * 16 + 1)
                lo = jnp.zeros((16,), jnp.int32)
                hi = jnp.full((16,), 1023, jnp.int32)

                def bs(_, carry):
                    lo2, hi2 = carry
                    mid = (lo2 + hi2) >> 1
                    val = plsc.load_gather(rstage, [qf, mid])
                    ok = val >= target
                    return jnp.where(ok, lo2, mid + 1), jnp.where(ok, mid, hi2)

                lo, hi = lax.fori_loop(0, 10, bs, (lo, hi))
                qoff = (sb * 16 + qi) * 32 + (sv & 1) * 16

                @pl.when(sv < 2)
                def _():
                    idxe[pl.ds(qoff, 16)] = lo + jb

                @pl.when(sv >= 2)
                def _():
                    idxo[pl.ds(qoff, 16)] = lo + jb

                return c2

            return lax.fori_loop(0, 4, slotv, c1)

        return lax.fori_loop(0, 16, perq, c0)

    lax.fori_loop(0, 8, sixteen, 0)

    def chunk(ci, c0):
        pltpu.async_copy(gl_hbm.at[idxe.at[pl.ds(ci * 512, 512)]], rows, sem).wait()
        pltpu.async_copy(gr_hbm.at[idxo.at[pl.ds(ci * 512, 512)]], rows, sem,
                         add=True).wait()
        pltpu.sync_copy(rows, out_hbm.at[pl.ds(qbase * 32 + ci * 512, 512)])
        return c0

    lax.fori_loop(0, 8, chunk, 0)


def _sc_gather(cs, gl, gr):
    mesh = plsc.VectorSubcoreMesh(core_axis_name="c", subcore_axis_name="s")
    f = functools.partial(
        pl.kernel,
        mesh=mesh,
        compiler_params=pltpu.CompilerParams(needs_layout_passes=False),
        out_type=jax.ShapeDtypeStruct((NQ * SP, 128), jnp.float32),
        scratch_types=[
            pltpu.VMEM((16, N), jnp.int32),
            pltpu.VMEM((4096,), jnp.int32),
            pltpu.VMEM((4096,), jnp.int32),
            pltpu.VMEM((512, 128), jnp.float32),
            pltpu.SemaphoreType.DMA,
        ],
    )(_sc_body)
    return f(cs, gl, gr)


# speedup vs baseline: 12.7927x; 1.0135x over previous
import functools

import jax
import jax.numpy as jnp
from jax import lax
from jax.experimental import pallas as pl
from jax.experimental.pallas import tpu as pltpu
from jax.experimental.pallas import tpu_sc as plsc

B, N, S, C = 4, 1024, 64, 64
EPS = 1e-3
CNT = float(B * N * S)
NQ = B * N
SP = S // 2
QB = 64
HI0 = 0x7F800000


def _k0_body(x1_ref, x2_ref, f1_ref, f2_ref, w1a_ref, w1b_ref, w1c_ref, b1_ref,
             gl_ref, gr_ref, h_ref):
    x1 = x1_ref[0]
    x2 = x2_ref[0]
    f1 = f1_ref[0]
    f2 = f2_ref[0]
    dn = (((0,), (0,)), ((), ()))
    g = (lax.dot_general(f2, w1b_ref[...], dn, preferred_element_type=jnp.float32)
         + lax.dot_general(x2, w1a_ref[...], dn, preferred_element_type=jnp.float32))
    h = (lax.dot_general(f1, w1c_ref[...], dn, preferred_element_type=jnp.float32)
         - lax.dot_general(x1, w1a_ref[...], dn, preferred_element_type=jnp.float32)
         + b1_ref[...])
    z = jnp.zeros((N, 64), jnp.float32)
    gl_ref[0] = jnp.concatenate([g, z], axis=1)
    gr_ref[0] = jnp.concatenate([z, g], axis=1)
    h_ref[0] = jnp.concatenate([h, h], axis=1)


def _cumsum0(x):
    xf = x.astype(jnp.float32)
    tri = (lax.broadcasted_iota(jnp.int32, (128, 128), 1)
           <= lax.broadcasted_iota(jnp.int32, (128, 128), 0)).astype(jnp.float32)
    parts = []
    off = jnp.zeros((1, 128), jnp.float32)
    for k in range(8):
        blk = lax.slice(xf, (k * 128, 0), (k * 128 + 128, 128))
        loc = lax.dot_general(tri, blk, (((1,), (0,)), ((), ())),
                              preferred_element_type=jnp.float32)
        parts.append(loc + off)
        off = off + lax.slice(loc, (127, 0), (128, 128))
    return jnp.concatenate(parts, axis=0).astype(jnp.int32)


def _ka_body(d_ref, r_ref):
    di = lax.bitcast_convert_type(d_ref[0], jnp.int32)

    def body(_, carry):
        lo, hi = carry
        mid = lo + ((hi - lo) >> 1)
        cnt = jnp.sum((di <= mid).astype(jnp.int32), axis=0, keepdims=True)
        ok = cnt >= S
        return jnp.where(ok, lo, mid), jnp.where(ok, mid, hi)

    lo0 = jnp.full((1, 128), -1, jnp.int32)
    hi0 = jnp.full((1, 128), HI0, jnp.int32)
    _, t = lax.fori_loop(0, 31, body, (lo0, hi0))

    lt = (di < t).astype(jnp.int32)
    c_lt = jnp.sum(lt, axis=0, keepdims=True)
    r_need = S - c_lt
    eq = (di == t).astype(jnp.int32)
    cs_eq = _cumsum0(eq)
    sel = lt + eq * ((cs_eq - eq) < r_need).astype(jnp.int32)
    r_ref[0] = _cumsum0(sel)


def _kc1_body(y_ref, h_ref, st_ref):
    i = pl.program_id(0)
    y = y_ref[...] + h_ref[...][:, None, :]
    s0 = jnp.sum(y, axis=(0, 1), keepdims=True)[0]
    s1 = jnp.sum(y * y, axis=(0, 1), keepdims=True)[0]
    st = jnp.concatenate([s0, s1], axis=0)

    @pl.when(i == 0)
    def _():
        st_ref[...] = st

    @pl.when(i > 0)
    def _():
        st_ref[...] += st


def _fold_stats(st):
    sf = st[:, :64] + st[:, 64:]
    sf = jnp.concatenate([sf, sf], axis=1)
    mean = sf[0:1] / CNT
    var = sf[1:2] / CNT - mean * mean
    inv = lax.rsqrt(var + EPS)
    return mean, inv


def _kc2_body(y_ref, h_ref, st_ref, p_ref, w_ref, b_ref, o_ref, st2_ref):
    i = pl.program_id(0)
    mean, inv = _fold_stats(st_ref[...])
    y = y_ref[...] + h_ref[...][:, None, :]
    x = jnp.reshape(y, (QB * SP, 128))
    x = jnp.maximum((x - mean) * inv * p_ref[0:1] + p_ref[1:2], 0.0)
    y2 = lax.dot_general(x, w_ref[...], (((1,), (0,)), ((), ())),
                         preferred_element_type=jnp.float32) + b_ref[...]
    o_ref[...] = jnp.reshape(y2, (QB, SP, 128))
    s0 = jnp.sum(y2, axis=0, keepdims=True)
    s1 = jnp.sum(y2 * y2, axis=0, keepdims=True)
    st = jnp.concatenate([s0, s1], axis=0)

    @pl.when(i == 0)
    def _():
        st2_ref[...] = st

    @pl.when(i > 0)
    def _():
        st2_ref[...] += st


def _kc3_body(y2_ref, st_ref, p_ref, w_ref, b_ref, mx_ref, mn_ref, st3_ref):
    i = pl.program_id(0)
    mean, inv = _fold_stats(st_ref[...])
    x = jnp.reshape(y2_ref[...], (QB * SP, 128))
    x = jnp.maximum((x - mean) * inv * p_ref[0:1] + p_ref[1:2], 0.0)
    y3 = lax.dot_general(x, w_ref[...], (((1,), (0,)), ((), ())),
                         preferred_element_type=jnp.float32) + b_ref[...]
    s0 = jnp.sum(y3, axis=0, keepdims=True)
    s1 = jnp.sum(y3 * y3, axis=0, keepdims=True)
    st = jnp.concatenate([s0, s1], axis=0)
    y3r = jnp.reshape(y3, (QB, SP, 128))
    mx = jnp.max(y3r, axis=1)
    mn = jnp.min(y3r, axis=1)
    mx_ref[...] = jnp.maximum(mx[:, :64], mx[:, 64:])
    mn_ref[...] = jnp.minimum(mn[:, :64], mn[:, 64:])

    @pl.when(i == 0)
    def _():
        st3_ref[...] = st

    @pl.when(i > 0)
    def _():
        st3_ref[...] += st


def _kc4_body(mx_ref, mn_ref, st_ref, g_ref, be_ref, o_ref):
    st = st_ref[...]
    sf = st[:, :64] + st[:, 64:]
    mean = sf[0:1] / CNT
    var = sf[1:2] / CNT - mean * mean
    inv = lax.rsqrt(var + EPS)
    g = g_ref[...]
    m = jnp.where(g > 0.0, mx_ref[...], mn_ref[...])
    o_ref[...] = jnp.maximum((m - mean) * inv * g + be_ref[...], 0.0)


def _tc_pipeline(xyz1p, xyz2p, feat1, feat2, w1a, w1b, w1c, b1r, dist):
    gl, gr, h = pl.pallas_call(
        _k0_body,
        grid=(B,),
        in_specs=[
            pl.BlockSpec((1, 8, N), lambda b: (b, 0, 0)),
            pl.BlockSpec((1, 8, N), lambda b: (b, 0, 0)),
            pl.BlockSpec((1, C, N), lambda b: (b, 0, 0)),
            pl.BlockSpec((1, C, N), lambda b: (b, 0, 0)),
            pl.BlockSpec((8, 64), lambda b: (0, 0)),
            pl.BlockSpec((C, 64), lambda b: (0, 0)),
            pl.BlockSpec((C, 64), lambda b: (0, 0)),
            pl.BlockSpec((1, 64), lambda b: (0, 0)),
        ],
        out_specs=[
            pl.BlockSpec((1, N, 128), lambda b: (b, 0, 0)),
            pl.BlockSpec((1, N, 128), lambda b: (b, 0, 0)),
            pl.BlockSpec((1, N, 128), lambda b: (b, 0, 0)),
        ],
        out_shape=[
            jax.ShapeDtypeStruct((B, N, 128), jnp.float32),
            jax.ShapeDtypeStruct((B, N, 128), jnp.float32),
            jax.ShapeDtypeStruct((B, N, 128), jnp.float32),
        ],
    )(xyz1p, xyz2p, feat1, feat2, w1a, w1b, w1c, b1r)

    r = pl.pallas_call(
        _ka_body,
        grid=(B, 8),
        in_specs=[
            pl.BlockSpec((1, N, 128), lambda b, q: (b, 0, q)),
        ],
        out_specs=pl.BlockSpec((1, N, 128), lambda b, q: (b, 0, q)),
        out_shape=jax.ShapeDtypeStruct((B, N, N), jnp.int32),
    )(dist)
    return gl, gr, h, r


def _tc_tail(y1g, hdup, p1, b2r, w2d, p2, b3r, w3d, g3r, be3r):
    grid = (NQ // QB,)
    st1 = pl.pallas_call(
        _kc1_body,
        grid=grid,
        in_specs=[
            pl.BlockSpec((QB, SP, 128), lambda i: (i, 0, 0)),
            pl.BlockSpec((QB, 128), lambda i: (i, 0)),
        ],
        out_specs=pl.BlockSpec((2, 128), lambda i: (0, 0)),
        out_shape=jax.ShapeDtypeStruct((2, 128), jnp.float32),
    )(y1g, hdup)

    y2, st2 = pl.pallas_call(
        _kc2_body,
        grid=grid,
        in_specs=[
            pl.BlockSpec((QB, SP, 128), lambda i: (i, 0, 0)),
            pl.BlockSpec((QB, 128), lambda i: (i, 0)),
            pl.BlockSpec((2, 128), lambda i: (0, 0)),
            pl.BlockSpec((2, 128), lambda i: (0, 0)),
            pl.BlockSpec((128, 128), lambda i: (0, 0)),
            pl.BlockSpec((1, 128), lambda i: (0, 0)),
        ],
        out_specs=[
            pl.BlockSpec((QB, SP, 128), lambda i: (i, 0, 0)),
            pl.BlockSpec((2, 128), lambda i: (0, 0)),
        ],
        out_shape=[
            jax.ShapeDtypeStruct((NQ, SP, 128), jnp.float32),
            jax.ShapeDtypeStruct((2, 128), jnp.float32),
        ],
    )(y1g, hdup, st1, p1, w2d, b2r)

    mx, mn, st3 = pl.pallas_call(
        _kc3_body,
        grid=grid,
        in_specs=[
            pl.BlockSpec((QB, SP, 128), lambda i: (i, 0, 0)),
            pl.BlockSpec((2, 128), lambda i: (0, 0)),
            pl.BlockSpec((2, 128), lambda i: (0, 0)),
            pl.BlockSpec((128, 128), lambda i: (0, 0)),
            pl.BlockSpec((1, 128), lambda i: (0, 0)),
        ],
        out_specs=[
            pl.BlockSpec((QB, 64), lambda i: (i, 0)),
            pl.BlockSpec((QB, 64), lambda i: (i, 0)),
            pl.BlockSpec((2, 128), lambda i: (0, 0)),
        ],
        out_shape=[
            jax.ShapeDtypeStruct((NQ, 64), jnp.float32),
            jax.ShapeDtypeStruct((NQ, 64), jnp.float32),
            jax.ShapeDtypeStruct((2, 128), jnp.float32),
        ],
    )(y2, st2, p2, w3d, b3r)

    out = pl.pallas_call(
        _kc4_body,
        grid=(1,),
        in_specs=[
            pl.BlockSpec((NQ, 64), lambda i: (0, 0)),
            pl.BlockSpec((NQ, 64), lambda i: (0, 0)),
            pl.BlockSpec((2, 128), lambda i: (0, 0)),
            pl.BlockSpec((1, 64), lambda i: (0, 0)),
            pl.BlockSpec((1, 64), lambda i: (0, 0)),
        ],
        out_specs=pl.BlockSpec((NQ, 64), lambda i: (0, 0)),
        out_shape=jax.ShapeDtypeStruct((NQ, 64), jnp.float32),
    )(mx, mn, st3, g3r, be3r)
    return out


def kernel(xyz1, xyz2, feat1, feat2, W1, b1, g1, be1, W2, b2, g2, be2, W3, b3, g3, be3):
    f32 = jnp.float32
    pad = jnp.zeros((B, 5, N), f32)
    xyz1p = jnp.concatenate([xyz1, pad], axis=1)
    xyz2p = jnp.concatenate([xyz2, pad], axis=1)
    w1a = jnp.concatenate([W1[:, :3], jnp.zeros((64, 5), f32)], axis=1).T
    w1b = W1[:, 3:67].T
    w1c = W1[:, 67:131].T
    b1r = b1[None, :]

    xx = jnp.sum(xyz2 ** 2, axis=1)[:, :, None]
    yy = jnp.sum(xyz1 ** 2, axis=1)[:, None, :]
    dist = xx + yy - 2.0 * jnp.einsum('bcn,bcm->bnm', xyz2, xyz1)
    dist = jnp.clip(jnp.nan_to_num(dist), 0.0, None)

    gl, gr, hdup, cs = _tc_pipeline(xyz1p, xyz2p, feat1, feat2, w1a, w1b, w1c,
                                    b1r, dist)

    cst = jnp.reshape(jnp.transpose(cs, (0, 2, 1)), (NQ, N))
    glf = jnp.reshape(gl, (NQ, 128))
    grf = jnp.reshape(gr, (NQ, 128))
    y1g = _sc_gather(cst, glf, grf)

    y1g = jnp.reshape(y1g, (NQ, SP, 128))
    hdupf = jnp.reshape(hdup, (NQ, 128))

    def dup(v):
        return jnp.concatenate([v, v])[None, :]

    p1 = jnp.concatenate([dup(g1), dup(be1)], axis=0)
    p2 = jnp.concatenate([dup(g2), dup(be2)], axis=0)
    w2t = W2.T
    w3t = W3.T
    zz = jnp.zeros((64, 64), f32)
    w2d = jnp.block([[w2t, zz], [zz, w2t]])
    w3d = jnp.block([[w3t, zz], [zz, w3t]])
    b2r = dup(b2)
    b3r = dup(b3)

    out = _tc_tail(y1g, hdupf, p1, b2r, w2d, p2, b3r, w3d, g3[None, :], be3[None, :])
    return jnp.transpose(jnp.reshape(out, (B, N, 64)), (0, 2, 1))


def _sc_body(cs_hbm, gl_hbm, gr_hbm, out_hbm, rstage, idxe, idxo, rows, rows2,
             sem, sem2):
    wid = lax.axis_index("s") * 2 + lax.axis_index("c")
    qbase = wid * 128

    def sixteen(sb, c0):
        base_q = qbase + sb * 16
        jb = ((base_q >> 10) << 10)
        pltpu.sync_copy(cs_hbm.at[pl.ds(base_q, 16)], rstage)

        def perq(qi, c1):
            qf = jnp.full((16,), qi, jnp.int32)

            def slotv(sv, c2):
                target = lax.iota(jnp.int32, 16) + (sv * 16 + 1)
                lo = jnp.zeros((16,), jnp.int32)
                hi = jnp.full((16,), 1023, jnp.int32)

                for _ in range(10):
                    mid = (lo + hi) >> 1
                    val = plsc.load_gather(rstage, [qf, mid])
                    ok = val >= target
                    lo = jnp.where(ok, lo, mid + 1)
                    hi = jnp.where(ok, mid, hi)
                qoff = (sb * 16 + qi) * 32 + (sv & 1) * 16

                @pl.when(sv < 2)
                def _():
                    idxe[pl.ds(qoff, 16)] = lo + jb

                @pl.when(sv >= 2)
                def _():
                    idxo[pl.ds(qoff, 16)] = lo + jb

                return c2

            return lax.fori_loop(0, 4, slotv, c1)

        return lax.fori_loop(0, 16, perq, c0)

    lax.fori_loop(0, 8, sixteen, 0)

    def left(ci, buf, sem):
        return pltpu.async_copy(gl_hbm.at[idxe.at[pl.ds(ci * 256, 256)]],
                                buf, sem)

    def right(ci, buf, sem):
        return pltpu.async_copy(gr_hbm.at[idxo.at[pl.ds(ci * 256, 256)]],
                                buf, sem, add=True)

    bufs = (rows, rows2)
    sems = (sem, sem2)
    lcp = [None, None]
    lcp[0] = left(0, bufs[0], sems[0])
    lcp[1] = left(1, bufs[1], sems[1])
    for ci in range(16):
        p = ci & 1
        lcp[p].wait()
        rcp = right(ci, bufs[p], sems[p])
        rcp.wait()
        pltpu.sync_copy(bufs[p], out_hbm.at[pl.ds(qbase * 32 + ci * 256, 256)])
        if ci + 2 < 16:
            lcp[p] = left(ci + 2, bufs[p], sems[p])


def _sc_gather(cs, gl, gr):
    mesh = plsc.VectorSubcoreMesh(core_axis_name="c", subcore_axis_name="s")
    f = functools.partial(
        pl.kernel,
        mesh=mesh,
        compiler_params=pltpu.CompilerParams(needs_layout_passes=False),
        out_type=jax.ShapeDtypeStruct((NQ * SP, 128), jnp.float32),
        scratch_types=[
            pltpu.VMEM((16, N), jnp.int32),
            pltpu.VMEM((4096,), jnp.int32),
            pltpu.VMEM((4096,), jnp.int32),
            pltpu.VMEM((256, 128), jnp.float32),
            pltpu.VMEM((256, 128), jnp.float32),
            pltpu.SemaphoreType.DMA,
            pltpu.SemaphoreType.DMA,
        ],
    )(_sc_body)
    return f(cs, gl, gr)


# speedup vs baseline: 13.6462x; 1.0667x over previous
import functools

import jax
import jax.numpy as jnp
from jax import lax
from jax.experimental import pallas as pl
from jax.experimental.pallas import tpu as pltpu
from jax.experimental.pallas import tpu_sc as plsc

B, N, S, C = 4, 1024, 64, 64
EPS = 1e-3
CNT = float(B * N * S)
NQ = B * N
SP = S // 2
QB = 64
HI0 = 0x7F800000


def _k0_body(x1_ref, x2_ref, f1_ref, f2_ref, w1a_ref, w1b_ref, w1c_ref, b1_ref,
             gl_ref, gr_ref, h_ref):
    x1 = x1_ref[0]
    x2 = x2_ref[0]
    f1 = f1_ref[0]
    f2 = f2_ref[0]
    dn = (((0,), (0,)), ((), ()))
    g = (lax.dot_general(f2, w1b_ref[...], dn, preferred_element_type=jnp.float32)
         + lax.dot_general(x2, w1a_ref[...], dn, preferred_element_type=jnp.float32))
    h = (lax.dot_general(f1, w1c_ref[...], dn, preferred_element_type=jnp.float32)
         - lax.dot_general(x1, w1a_ref[...], dn, preferred_element_type=jnp.float32)
         + b1_ref[...])
    z = jnp.zeros((N, 64), jnp.float32)
    gl_ref[0] = jnp.concatenate([g, z], axis=1)
    gr_ref[0] = jnp.concatenate([z, g], axis=1)
    h_ref[0] = jnp.concatenate([h, h], axis=1)


def _cumsum0(x):
    xf = x.astype(jnp.float32)
    tri = (lax.broadcasted_iota(jnp.int32, (128, 128), 1)
           <= lax.broadcasted_iota(jnp.int32, (128, 128), 0)).astype(jnp.float32)
    parts = []
    off = jnp.zeros((1, 128), jnp.float32)
    for k in range(8):
        blk = lax.slice(xf, (k * 128, 0), (k * 128 + 128, 128))
        loc = lax.dot_general(tri, blk, (((1,), (0,)), ((), ())),
                              preferred_element_type=jnp.float32)
        parts.append(loc + off)
        off = off + lax.slice(loc, (127, 0), (128, 128))
    return jnp.concatenate(parts, axis=0).astype(jnp.int32)


def _ka_body(d_ref, r_ref):
    di = lax.bitcast_convert_type(d_ref[0], jnp.int32)

    def body(_, carry):
        lo, hi = carry
        mid = lo + ((hi - lo) >> 1)
        cnt = jnp.sum((di <= mid).astype(jnp.int32), axis=0, keepdims=True)
        ok = cnt >= S
        return jnp.where(ok, lo, mid), jnp.where(ok, mid, hi)

    lo0 = jnp.full((1, 128), -1, jnp.int32)
    hi0 = jnp.full((1, 128), HI0, jnp.int32)
    _, t = lax.fori_loop(0, 31, body, (lo0, hi0))

    lt = (di < t).astype(jnp.int32)
    c_lt = jnp.sum(lt, axis=0, keepdims=True)
    r_need = S - c_lt
    eq = (di == t).astype(jnp.int32)
    cs_eq = _cumsum0(eq)
    sel = lt + eq * ((cs_eq - eq) < r_need).astype(jnp.int32)
    r_ref[0] = _cumsum0(sel)


def _kc1_body(y_ref, h_ref, st_ref):
    i = pl.program_id(0)
    y = y_ref[...] + h_ref[...][:, None, :]
    s0 = jnp.sum(y, axis=(0, 1), keepdims=True)[0]
    s1 = jnp.sum(y * y, axis=(0, 1), keepdims=True)[0]
    st = jnp.concatenate([s0, s1], axis=0)

    @pl.when(i == 0)
    def _():
        st_ref[...] = st

    @pl.when(i > 0)
    def _():
        st_ref[...] += st


def _fold_stats(st):
    sf = st[:, :64] + st[:, 64:]
    sf = jnp.concatenate([sf, sf], axis=1)
    mean = sf[0:1] / CNT
    var = sf[1:2] / CNT - mean * mean
    inv = lax.rsqrt(var + EPS)
    return mean, inv


def _kc2_body(y_ref, h_ref, st_ref, p_ref, w_ref, b_ref, o_ref, st2_ref):
    i = pl.program_id(0)
    mean, inv = _fold_stats(st_ref[...])
    y = y_ref[...] + h_ref[...][:, None, :]
    x = jnp.reshape(y, (QB * SP, 128))
    x = jnp.maximum((x - mean) * inv * p_ref[0:1] + p_ref[1:2], 0.0)
    y2 = lax.dot_general(x, w_ref[...], (((1,), (0,)), ((), ())),
                         preferred_element_type=jnp.float32) + b_ref[...]
    o_ref[...] = jnp.reshape(y2, (QB, SP, 128))
    s0 = jnp.sum(y2, axis=0, keepdims=True)
    s1 = jnp.sum(y2 * y2, axis=0, keepdims=True)
    st = jnp.concatenate([s0, s1], axis=0)

    @pl.when(i == 0)
    def _():
        st2_ref[...] = st

    @pl.when(i > 0)
    def _():
        st2_ref[...] += st


def _kc3_body(y2_ref, st_ref, p_ref, w_ref, b_ref, mx_ref, mn_ref, st3_ref):
    i = pl.program_id(0)
    mean, inv = _fold_stats(st_ref[...])
    x = jnp.reshape(y2_ref[...], (QB * SP, 128))
    x = jnp.maximum((x - mean) * inv * p_ref[0:1] + p_ref[1:2], 0.0)
    y3 = lax.dot_general(x, w_ref[...], (((1,), (0,)), ((), ())),
                         preferred_element_type=jnp.float32) + b_ref[...]
    s0 = jnp.sum(y3, axis=0, keepdims=True)
    s1 = jnp.sum(y3 * y3, axis=0, keepdims=True)
    st = jnp.concatenate([s0, s1], axis=0)
    y3r = jnp.reshape(y3, (QB, SP, 128))
    mx = jnp.max(y3r, axis=1)
    mn = jnp.min(y3r, axis=1)
    mx_ref[...] = jnp.maximum(mx[:, :64], mx[:, 64:])
    mn_ref[...] = jnp.minimum(mn[:, :64], mn[:, 64:])

    @pl.when(i == 0)
    def _():
        st3_ref[...] = st

    @pl.when(i > 0)
    def _():
        st3_ref[...] += st


def _kc4_body(mx_ref, mn_ref, st_ref, g_ref, be_ref, o_ref):
    st = st_ref[...]
    sf = st[:, :64] + st[:, 64:]
    mean = sf[0:1] / CNT
    var = sf[1:2] / CNT - mean * mean
    inv = lax.rsqrt(var + EPS)
    g = g_ref[...]
    m = jnp.where(g > 0.0, mx_ref[...], mn_ref[...])
    o_ref[...] = jnp.maximum((m - mean) * inv * g + be_ref[...], 0.0)


def _tc_pipeline(xyz1p, xyz2p, feat1, feat2, w1a, w1b, w1c, b1r, dist):
    gl, gr, h = pl.pallas_call(
        _k0_body,
        grid=(B,),
        in_specs=[
            pl.BlockSpec((1, 8, N), lambda b: (b, 0, 0)),
            pl.BlockSpec((1, 8, N), lambda b: (b, 0, 0)),
            pl.BlockSpec((1, C, N), lambda b: (b, 0, 0)),
            pl.BlockSpec((1, C, N), lambda b: (b, 0, 0)),
            pl.BlockSpec((8, 64), lambda b: (0, 0)),
            pl.BlockSpec((C, 64), lambda b: (0, 0)),
            pl.BlockSpec((C, 64), lambda b: (0, 0)),
            pl.BlockSpec((1, 64), lambda b: (0, 0)),
        ],
        out_specs=[
            pl.BlockSpec((1, N, 128), lambda b: (b, 0, 0)),
            pl.BlockSpec((1, N, 128), lambda b: (b, 0, 0)),
            pl.BlockSpec((1, N, 128), lambda b: (b, 0, 0)),
        ],
        out_shape=[
            jax.ShapeDtypeStruct((B, N, 128), jnp.float32),
            jax.ShapeDtypeStruct((B, N, 128), jnp.float32),
            jax.ShapeDtypeStruct((B, N, 128), jnp.float32),
        ],
    )(xyz1p, xyz2p, feat1, feat2, w1a, w1b, w1c, b1r)

    r = pl.pallas_call(
        _ka_body,
        grid=(B, 8),
        in_specs=[
            pl.BlockSpec((1, N, 128), lambda b, q: (b, 0, q)),
        ],
        out_specs=pl.BlockSpec((1, N, 128), lambda b, q: (b, 0, q)),
        out_shape=jax.ShapeDtypeStruct((B, N, N), jnp.int32),
    )(dist)
    return gl, gr, h, r


def _tc_tail(y1g, hdup, p1, b2r, w2d, p2, b3r, w3d, g3r, be3r):
    grid = (NQ // QB,)
    st1 = pl.pallas_call(
        _kc1_body,
        grid=grid,
        in_specs=[
            pl.BlockSpec((QB, SP, 128), lambda i: (i, 0, 0)),
            pl.BlockSpec((QB, 128), lambda i: (i, 0)),
        ],
        out_specs=pl.BlockSpec((2, 128), lambda i: (0, 0)),
        out_shape=jax.ShapeDtypeStruct((2, 128), jnp.float32),
    )(y1g, hdup)

    y2, st2 = pl.pallas_call(
        _kc2_body,
        grid=grid,
        in_specs=[
            pl.BlockSpec((QB, SP, 128), lambda i: (i, 0, 0)),
            pl.BlockSpec((QB, 128), lambda i: (i, 0)),
            pl.BlockSpec((2, 128), lambda i: (0, 0)),
            pl.BlockSpec((2, 128), lambda i: (0, 0)),
            pl.BlockSpec((128, 128), lambda i: (0, 0)),
            pl.BlockSpec((1, 128), lambda i: (0, 0)),
        ],
        out_specs=[
            pl.BlockSpec((QB, SP, 128), lambda i: (i, 0, 0)),
            pl.BlockSpec((2, 128), lambda i: (0, 0)),
        ],
        out_shape=[
            jax.ShapeDtypeStruct((NQ, SP, 128), jnp.float32),
            jax.ShapeDtypeStruct((2, 128), jnp.float32),
        ],
    )(y1g, hdup, st1, p1, w2d, b2r)

    mx, mn, st3 = pl.pallas_call(
        _kc3_body,
        grid=grid,
        in_specs=[
            pl.BlockSpec((QB, SP, 128), lambda i: (i, 0, 0)),
            pl.BlockSpec((2, 128), lambda i: (0, 0)),
            pl.BlockSpec((2, 128), lambda i: (0, 0)),
            pl.BlockSpec((128, 128), lambda i: (0, 0)),
            pl.BlockSpec((1, 128), lambda i: (0, 0)),
        ],
        out_specs=[
            pl.BlockSpec((QB, 64), lambda i: (i, 0)),
            pl.BlockSpec((QB, 64), lambda i: (i, 0)),
            pl.BlockSpec((2, 128), lambda i: (0, 0)),
        ],
        out_shape=[
            jax.ShapeDtypeStruct((NQ, 64), jnp.float32),
            jax.ShapeDtypeStruct((NQ, 64), jnp.float32),
            jax.ShapeDtypeStruct((2, 128), jnp.float32),
        ],
    )(y2, st2, p2, w3d, b3r)

    out = pl.pallas_call(
        _kc4_body,
        grid=(1,),
        in_specs=[
            pl.BlockSpec((NQ, 64), lambda i: (0, 0)),
            pl.BlockSpec((NQ, 64), lambda i: (0, 0)),
            pl.BlockSpec((2, 128), lambda i: (0, 0)),
            pl.BlockSpec((1, 64), lambda i: (0, 0)),
            pl.BlockSpec((1, 64), lambda i: (0, 0)),
        ],
        out_specs=pl.BlockSpec((NQ, 64), lambda i: (0, 0)),
        out_shape=jax.ShapeDtypeStruct((NQ, 64), jnp.float32),
    )(mx, mn, st3, g3r, be3r)
    return out


def kernel(xyz1, xyz2, feat1, feat2, W1, b1, g1, be1, W2, b2, g2, be2, W3, b3, g3, be3):
    f32 = jnp.float32
    pad = jnp.zeros((B, 5, N), f32)
    xyz1p = jnp.concatenate([xyz1, pad], axis=1)
    xyz2p = jnp.concatenate([xyz2, pad], axis=1)
    w1a = jnp.concatenate([W1[:, :3], jnp.zeros((64, 5), f32)], axis=1).T
    w1b = W1[:, 3:67].T
    w1c = W1[:, 67:131].T
    b1r = b1[None, :]

    xx = jnp.sum(xyz2 ** 2, axis=1)[:, :, None]
    yy = jnp.sum(xyz1 ** 2, axis=1)[:, None, :]
    dist = xx + yy - 2.0 * jnp.einsum('bcn,bcm->bnm', xyz2, xyz1)
    dist = jnp.clip(jnp.nan_to_num(dist), 0.0, None)

    gl, gr, hdup, cs = _tc_pipeline(xyz1p, xyz2p, feat1, feat2, w1a, w1b, w1c,
                                    b1r, dist)

    cst = jnp.reshape(jnp.transpose(cs, (0, 2, 1)), (NQ, N))
    glf = jnp.reshape(gl, (NQ, 128))
    grf = jnp.reshape(gr, (NQ, 128))
    y1g = _sc_gather(cst, glf, grf)

    y1g = jnp.reshape(y1g, (NQ, SP, 128))
    hdupf = jnp.reshape(hdup, (NQ, 128))

    def dup(v):
        return jnp.concatenate([v, v])[None, :]

    p1 = jnp.concatenate([dup(g1), dup(be1)], axis=0)
    p2 = jnp.concatenate([dup(g2), dup(be2)], axis=0)
    w2t = W2.T
    w3t = W3.T
    zz = jnp.zeros((64, 64), f32)
    w2d = jnp.block([[w2t, zz], [zz, w2t]])
    w3d = jnp.block([[w3t, zz], [zz, w3t]])
    b2r = dup(b2)
    b3r = dup(b3)

    out = _tc_tail(y1g, hdupf, p1, b2r, w2d, p2, b3r, w3d, g3[None, :], be3[None, :])
    return jnp.transpose(jnp.reshape(out, (B, N, 64)), (0, 2, 1))


def _sc_body(cs_hbm, gl_hbm, gr_hbm, out_hbm, rstage, idxe, idxo, rows, rows2,
             sem, sem2):
    wid = lax.axis_index("s") * 2 + lax.axis_index("c")
    qbase = wid * 128

    def sixteen(sb, c0):
        base_q = qbase + sb * 16
        jb = ((base_q >> 10) << 10)
        pltpu.sync_copy(cs_hbm.at[pl.ds(base_q, 16)], rstage)

        def perq(qp, c1):
            chains = []
            for u in range(2):
                qi = qp * 2 + u
                qf = jnp.full((16,), qi, jnp.int32)
                for sv in range(4):
                    target = lax.iota(jnp.int32, 16) + (sv * 16 + 1)
                    chains.append([qf, target, jnp.zeros((16,), jnp.int32),
                                   jnp.full((16,), 1023, jnp.int32), u, sv])
            for _ in range(10):
                mids = [(ch[2] + ch[3]) >> 1 for ch in chains]
                vals = [plsc.load_gather(rstage, [ch[0], mid])
                        for ch, mid in zip(chains, mids)]
                for ch, mid, val in zip(chains, mids, vals):
                    ok = val >= ch[1]
                    ch[2] = jnp.where(ok, ch[2], mid + 1)
                    ch[3] = jnp.where(ok, mid, ch[3])
            for ch in chains:
                u, sv = ch[4], ch[5]
                qoff = (sb * 16 + qp * 2 + u) * 32 + (sv & 1) * 16
                if sv < 2:
                    idxe[pl.ds(qoff, 16)] = ch[3] + jb
                else:
                    idxo[pl.ds(qoff, 16)] = ch[3] + jb
            return c1

        return lax.fori_loop(0, 8, perq, c0)

    lax.fori_loop(0, 8, sixteen, 0)

    def left(ci, buf, sem):
        return pltpu.async_copy(gl_hbm.at[idxe.at[pl.ds(ci * 256, 256)]],
                                buf, sem)

    def right(ci, buf, sem):
        return pltpu.async_copy(gr_hbm.at[idxo.at[pl.ds(ci * 256, 256)]],
                                buf, sem, add=True)

    bufs = (rows, rows2)
    sems = (sem, sem2)
    lcp = [None, None]
    lcp[0] = left(0, bufs[0], sems[0])
    lcp[1] = left(1, bufs[1], sems[1])
    for ci in range(16):
        p = ci & 1
        lcp[p].wait()
        rcp = right(ci, bufs[p], sems[p])
        rcp.wait()
        pltpu.sync_copy(bufs[p], out_hbm.at[pl.ds(qbase * 32 + ci * 256, 256)])
        if ci + 2 < 16:
            lcp[p] = left(ci + 2, bufs[p], sems[p])


def _sc_gather(cs, gl, gr):
    mesh = plsc.VectorSubcoreMesh(core_axis_name="c", subcore_axis_name="s")
    f = functools.partial(
        pl.kernel,
        mesh=mesh,
        compiler_params=pltpu.CompilerParams(needs_layout_passes=False),
        out_type=jax.ShapeDtypeStruct((NQ * SP, 128), jnp.float32),
        scratch_types=[
            pltpu.VMEM((16, N), jnp.int32),
            pltpu.VMEM((4096,), jnp.int32),
            pltpu.VMEM((4096,), jnp.int32),
            pltpu.VMEM((256, 128), jnp.float32),
            pltpu.VMEM((256, 128), jnp.float32),
            pltpu.SemaphoreType.DMA,
            pltpu.SemaphoreType.DMA,
        ],
    )(_sc_body)
    return f(cs, gl, gr)
